# Initial kernel scaffold; baseline (speedup 1.0000x reference)
import jax
import jax.numpy as jnp
from jax.experimental import pallas as pl


def kernel(x, pos, support_points, indices, W1, b1, W2, b2):
    raise NotImplementedError("write your pallas kernel here")


# speedup vs baseline: 42.8393x; 42.8393x over previous
import functools

import jax
import jax.numpy as jnp
from jax import lax
from jax.experimental import pallas as pl
from jax.experimental.pallas import tpu as pltpu
from jax.experimental.pallas import tpu_sc as plsc


def _ztab_body(x_ref, pos_ref, w1x_ref, w1p_ref, z_ref):
    x = x_ref[0]
    p = pos_ref[0]
    zx = lax.dot_general(x, w1x_ref[...], (((0,), (1,)), ((), ())),
                         preferred_element_type=jnp.float32)
    zp = lax.dot_general(p, w1p_ref[...], (((0,), (1,)), ((), ())),
                         preferred_element_type=jnp.float32)
    z_ref[0] = zx + zp


def _make_ztab(B, C, N, H1):
    return pl.pallas_call(
        _ztab_body,
        grid=(B,),
        in_specs=[
            pl.BlockSpec((1, C, N), lambda b: (b, 0, 0)),
            pl.BlockSpec((1, 3, N), lambda b: (b, 0, 0)),
            pl.BlockSpec((H1, C), lambda b: (0, 0)),
            pl.BlockSpec((H1, 3), lambda b: (0, 0)),
        ],
        out_specs=pl.BlockSpec((1, N, H1), lambda b: (b, 0, 0)),
        out_shape=jax.ShapeDtypeStruct((B, N, H1), jnp.float32),
    )


def _make_sc_gather(BR, H1, IDXW):
    info = plsc.get_sparse_core_info()
    NC, NS = info.num_cores, info.num_subcores
    NW = NC * NS
    rows_w = BR // NW
    SUB = 8
    CHUNK = SUB * IDXW
    nchunk = rows_w // CHUNK
    assert rows_w % CHUNK == 0

    mesh = plsc.VectorSubcoreMesh(core_axis_name="c", subcore_axis_name="s")

    @functools.partial(
        pl.kernel,
        mesh=mesh,
        out_type=jax.ShapeDtypeStruct((BR, H1), jnp.float32),
        scratch_types=[
            pltpu.VMEM((SUB, IDXW), jnp.int32),
            pltpu.VMEM((CHUNK, H1), jnp.float32),
            pltpu.SemaphoreType.DMA,
        ],
        compiler_params=pltpu.CompilerParams(use_tc_tiling_on_sc=False),
    )
    def k(tab_hbm, idx_hbm, out_hbm, idx_v, rows_v, sem):
        wid = lax.axis_index("s") * NC + lax.axis_index("c")
        base = wid * rows_w
        ibase = wid * (rows_w // IDXW)

        def chunk(g, carry):
            off = base + g * CHUNK
            pltpu.sync_copy(idx_hbm.at[pl.ds(ibase + g * SUB, SUB)], idx_v)
            handles = []
            for j in range(SUB):
                handles.append(
                    pltpu.async_copy(tab_hbm.at[idx_v.at[j]],
                                     rows_v.at[pl.ds(j * IDXW, IDXW)], sem))
            for h in handles:
                h.wait()
            pltpu.sync_copy(rows_v, out_hbm.at[pl.ds(off, CHUNK)])
            return carry

        lax.fori_loop(0, nchunk, chunk, 0)

    return k


def _head_body(K, MB, g_ref, sup_ref, w1p_ref, b1_ref, w2_ref, b2_ref, o_ref):
    H1 = g_ref.shape[2]
    t = lax.dot_general(sup_ref[0], w1p_ref[...], (((1,), (1,)), ((), ())),
                        preferred_element_type=jnp.float32) - b1_ref[...]
    g3 = g_ref[0].reshape(MB, K, H1)
    r = jnp.maximum(g3 - t[:, None, :], 0.0)
    h = lax.dot_general(r.reshape(MB * K, H1), w2_ref[...],
                        (((1,), (1,)), ((), ())),
                        preferred_element_type=jnp.float32)
    OUT = h.shape[1]
    o = jnp.max(h.reshape(MB, K, OUT), axis=1)
    o_ref[0] = o + b2_ref[...]


def _make_head(B, M, K, H1, OUT, MB):
    nmb = M // MB
    return pl.pallas_call(
        functools.partial(_head_body, K, MB),
        grid=(B, nmb),
        in_specs=[
            pl.BlockSpec((1, MB * K, H1), lambda b, i: (b, i, 0)),
            pl.BlockSpec((1, MB, 3), lambda b, i: (b, i, 0)),
            pl.BlockSpec((H1, 3), lambda b, i: (0, 0)),
            pl.BlockSpec((1, H1), lambda b, i: (0, 0)),
            pl.BlockSpec((OUT, H1), lambda b, i: (0, 0)),
            pl.BlockSpec((1, OUT), lambda b, i: (0, 0)),
        ],
        out_specs=pl.BlockSpec((1, MB, OUT), lambda b, i: (b, i, 0)),
        out_shape=jax.ShapeDtypeStruct((B, M, OUT), jnp.float32),
        compiler_params=pltpu.CompilerParams(
            dimension_semantics=("parallel", "parallel")),
    )


def kernel(x, pos, support_points, indices, W1, b1, W2, b2):
    B, C, N = x.shape
    _, M, K = indices.shape
    H1 = W1.shape[0]
    OUT = W2.shape[0]
    W1x = W1[:, :C]
    W1p = W1[:, C:]

    z = _make_ztab(B, C, N, H1)(x, pos, W1x, W1p)

    BR = B * M * K
    IDXW = 125
    idxf = (indices.reshape(B, M * K)
            + (jnp.arange(B, dtype=jnp.int32) * N)[:, None]).reshape(BR // IDXW, IDXW)
    g = _make_sc_gather(BR, H1, IDXW)(z.reshape(B * N, H1), idxf)

    supT = support_points.transpose(0, 2, 1)
    out = _make_head(B, M, K, H1, OUT, MB=200)(
        g.reshape(B, M * K, H1), supT, W1p,
        b1.reshape(1, H1), W2, b2.reshape(1, OUT))
    return out.transpose(0, 2, 1)


# speedup vs baseline: 68.9180x; 1.6088x over previous
import functools

import jax
import jax.numpy as jnp
from jax import lax
from jax.experimental import pallas as pl
from jax.experimental.pallas import tpu as pltpu
from jax.experimental.pallas import tpu_sc as plsc


def _ztab_body(x_ref, pos_ref, w1x_ref, w1p_ref, z_ref):
    x = x_ref[0]
    p = pos_ref[0]
    zx = lax.dot_general(x, w1x_ref[...], (((0,), (1,)), ((), ())),
                         preferred_element_type=jnp.float32)
    zp = lax.dot_general(p, w1p_ref[...], (((0,), (1,)), ((), ())),
                         preferred_element_type=jnp.float32)
    z_ref[0] = zx + zp


def _make_ztab(B, C, N, H1):
    return pl.pallas_call(
        _ztab_body,
        grid=(B,),
        in_specs=[
            pl.BlockSpec((1, C, N), lambda b: (b, 0, 0)),
            pl.BlockSpec((1, 3, N), lambda b: (b, 0, 0)),
            pl.BlockSpec((H1, C), lambda b: (0, 0)),
            pl.BlockSpec((H1, 3), lambda b: (0, 0)),
        ],
        out_specs=pl.BlockSpec((1, N, H1), lambda b: (b, 0, 0)),
        out_shape=jax.ShapeDtypeStruct((B, N, H1), jnp.float32),
    )


def _make_sc_gather(BR, H1, IDXW):
    info = plsc.get_sparse_core_info()
    NC, NS = info.num_cores, info.num_subcores
    NW = NC * NS
    rows_w = BR // NW
    SUB = 8
    CHUNK = SUB * IDXW
    nchunk = rows_w // CHUNK
    assert rows_w % CHUNK == 0
    PK = 128 // H1
    assert CHUNK % PK == 0 and BR % PK == 0

    mesh = plsc.VectorSubcoreMesh(core_axis_name="c", subcore_axis_name="s")

    @functools.partial(
        pl.kernel,
        mesh=mesh,
        out_type=jax.ShapeDtypeStruct((BR, H1), jnp.float32),
        scratch_types=[
            pltpu.VMEM((SUB, IDXW), jnp.int32),
            pltpu.VMEM((CHUNK, H1), jnp.float32),
            pltpu.SemaphoreType.DMA,
        ],
        compiler_params=pltpu.CompilerParams(use_tc_tiling_on_sc=False),
    )
    def k(tab_hbm, idx_hbm, out_hbm, idx_v, rows_v, sem):
        wid = lax.axis_index("s") * NC + lax.axis_index("c")
        base = wid * rows_w
        ibase = wid * (rows_w // IDXW)

        def chunk(g, carry):
            pltpu.sync_copy(idx_hbm.at[pl.ds(ibase + g * SUB, SUB)], idx_v)
            handles = []
            for j in range(SUB):
                handles.append(
                    pltpu.async_copy(tab_hbm.at[idx_v.at[j]],
                                     rows_v.at[pl.ds(j * IDXW, IDXW)], sem))
            for h in handles:
                h.wait()
            pltpu.sync_copy(rows_v,
                            out_hbm.at[pl.ds(base + g * CHUNK, CHUNK)])
            return carry

        lax.fori_loop(0, nchunk, chunk, 0)

    return k


def _head_body(K, MB, PK, g_ref, sup_ref, w1p_ref, b1_ref, w2b_ref, b2_ref,
               o_ref):
    H1 = w1p_ref.shape[0]
    KP = K // PK
    t = lax.dot_general(sup_ref[0], w1p_ref[...], (((1,), (1,)), ((), ())),
                        preferred_element_type=jnp.float32) - b1_ref[...]
    t4 = jnp.concatenate([t] * PK, axis=1)
    g3 = g_ref[0].reshape(KP, MB, PK * H1)
    r = jnp.maximum(g3 - t4[None, :, :], 0.0).reshape(KP * MB, PK * H1)
    h = lax.dot_general(r, w2b_ref[...], (((1,), (0,)), ((), ())),
                        preferred_element_type=jnp.float32)
    OUT = o_ref.shape[2]
    h3 = h.reshape(KP, MB, PK * OUT)
    hm = jnp.max(h3, axis=0)
    o = hm[:, :OUT]
    for q in range(1, PK):
        o = jnp.maximum(o, hm[:, q * OUT:(q + 1) * OUT])
    o_ref[0] = o + b2_ref[...]


def _make_head(B, M, K, H1, OUT, MB, PK):
    nmb = M // MB
    RP = M * K // PK
    return pl.pallas_call(
        functools.partial(_head_body, K, MB, PK),
        grid=(B, nmb),
        in_specs=[
            pl.BlockSpec((1, MB * K // PK, PK * H1), lambda b, i: (b, i, 0)),
            pl.BlockSpec((1, MB, 3), lambda b, i: (b, i, 0)),
            pl.BlockSpec((H1, 3), lambda b, i: (0, 0)),
            pl.BlockSpec((1, H1), lambda b, i: (0, 0)),
            pl.BlockSpec((PK * H1, PK * OUT), lambda b, i: (0, 0)),
            pl.BlockSpec((1, OUT), lambda b, i: (0, 0)),
        ],
        out_specs=pl.BlockSpec((1, MB, OUT), lambda b, i: (b, i, 0)),
        out_shape=jax.ShapeDtypeStruct((B, M, OUT), jnp.float32),
        compiler_params=pltpu.CompilerParams(
            dimension_semantics=("parallel", "parallel")),
    )


def kernel(x, pos, support_points, indices, W1, b1, W2, b2):
    B, C, N = x.shape
    _, M, K = indices.shape
    H1 = W1.shape[0]
    OUT = W2.shape[0]
    W1x = W1[:, :C]
    W1p = W1[:, C:]

    z = _make_ztab(B, C, N, H1)(x, pos, W1x, W1p)

    BR = B * M * K
    IDXW = 100
    PK = 128 // H1
    MB = 400
    KP = K // PK
    idxr = (indices.reshape(B, M // MB, MB, KP, PK)
            .transpose(0, 1, 3, 2, 4)
            .reshape(B, M * K))
    idxf = (idxr + (jnp.arange(B, dtype=jnp.int32) * N)[:, None]
            ).reshape(BR // IDXW, IDXW)
    g = _make_sc_gather(BR, H1, IDXW)(z.reshape(B * N, H1), idxf)

    supT = support_points.transpose(0, 2, 1)
    W2blk = jnp.kron(jnp.eye(PK, dtype=W2.dtype), W2.T)
    out = _make_head(B, M, K, H1, OUT, MB=MB, PK=PK)(
        g.reshape(B, M * K // PK, PK * H1), supT, W1p,
        b1.reshape(1, H1), W2blk, b2.reshape(1, OUT))
    return out.transpose(0, 2, 1)
